# baseline (speedup 1.0000x reference)
import functools

import jax
import jax.numpy as jnp
from jax import lax
from jax.experimental import pallas as pl
from jax.experimental.pallas import tpu as pltpu
from jax.experimental.pallas import tpu_sc as plsc

NUM_REL = 16
DIM = 64
N_EDGES = 800000

_info = plsc.get_sparse_core_info()
_NC, _NS = _info.num_cores, _info.num_subcores
_NW = _NC * _NS
_B_PER_W = N_EDGES // _NW
_CHUNK = 200
_N_STEPS = _B_PER_W // _CHUNK
_FULL_GROUPS = _CHUNK // 16
_TAIL_START = _CHUNK - 16
_N_PAIRS = (_N_STEPS - 1) // 2


def _make_sc_kernel():
    mesh = plsc.VectorSubcoreMesh(core_axis_name="c", subcore_axis_name="s")

    @functools.partial(
        pl.kernel,
        mesh=mesh,
        compiler_params=pltpu.CompilerParams(
            use_tc_tiling_on_sc=False, needs_layout_passes=False
        ),
        out_type=jax.ShapeDtypeStruct((N_EDGES * DIM,), jnp.float32),
        scratch_types=[
            pltpu.VMEM((_B_PER_W,), jnp.int32),
            pltpu.VMEM((_CHUNK * DIM,), jnp.float32),
            pltpu.VMEM((_CHUNK * DIM,), jnp.float32),
            pltpu.VMEM((NUM_REL * DIM,), jnp.float32),
            pltpu.SemaphoreType.DMA,
            pltpu.SemaphoreType.DMA,
        ],
    )
    def k(idx_hbm, table_hbm, out_hbm, idx_v, rows0, rows1, table_v, sw0, sw1):
        wid = lax.axis_index("s") * _NC + lax.axis_index("c")
        base = wid * _B_PER_W
        rows = (rows0, rows1)
        sw = (sw0, sw1)

        pltpu.sync_copy(table_hbm, table_v)
        pltpu.sync_copy(idx_hbm.at[pl.ds(base, _B_PER_W)], idx_v)
        lane64 = lax.iota(jnp.int32, 16) * DIM

        def compute_chunk(i, rows_ref):
            def group(local_start):
                idxv = idx_v[pl.ds(i * _CHUNK + local_start, 16)]
                row_base = idxv << 6
                dstb = lane64 + local_start * DIM
                for c in range(DIM):
                    vals = plsc.load_gather(table_v, [row_base + c])
                    plsc.store_scatter(rows_ref, [dstb + c], vals)

            def body(j, carry):
                group(j * 16)
                return carry

            lax.fori_loop(0, _FULL_GROUPS, body, 0)
            group(_TAIL_START)

        def writeback(i, b):
            off = (base + i * _CHUNK) * DIM
            pltpu.async_copy(rows[b], out_hbm.at[pl.ds(off, _CHUNK * DIM)],
                             sw[b])

        def drain(i, b):
            off = (base + i * _CHUNK) * DIM
            pltpu.make_async_copy(
                rows[b], out_hbm.at[pl.ds(off, _CHUNK * DIM)], sw[b]
            ).wait()

        compute_chunk(0, rows0)
        writeback(0, 0)

        def pair(it, carry):
            i1 = 1 + 2 * it
            i0 = 2 + 2 * it

            @pl.when(it > 0)
            def _():
                drain(i1 - 2, 1)

            compute_chunk(i1, rows1)
            writeback(i1, 1)
            drain(i0 - 2, 0)
            compute_chunk(i0, rows0)
            writeback(i0, 0)
            return carry

        lax.fori_loop(0, _N_PAIRS, pair, 0)
        drain(_N_STEPS - 2, 1)
        drain(_N_STEPS - 1, 0)

    return k


_sc_kernel = _make_sc_kernel()


def kernel(relation_indices, W):
    idx = relation_indices.astype(jnp.int32)
    flat = _sc_kernel(idx, jnp.reshape(W, (-1,)))
    return jnp.reshape(flat, (N_EDGES, DIM))


# speedup vs baseline: 1.3313x; 1.3313x over previous
import functools

import jax
import jax.numpy as jnp
from jax import lax
from jax.experimental import pallas as pl
from jax.experimental.pallas import tpu as pltpu
from jax.experimental.pallas import tpu_sc as plsc

NUM_REL = 16
DIM = 64
N_EDGES = 800000

_info = plsc.get_sparse_core_info()
_NC, _NS = _info.num_cores, _info.num_subcores
_NW = _NC * _NS
_B_PER_W = N_EDGES // _NW
_CHUNK = 200
_N_STEPS = _B_PER_W // _CHUNK
_FULL_GROUPS = _CHUNK // 16
_TAIL_START = _CHUNK - 16
_N_PAIRS = (_N_STEPS - 1) // 2


def _make_sc_kernel():
    mesh = plsc.VectorSubcoreMesh(core_axis_name="c", subcore_axis_name="s")

    @functools.partial(
        pl.kernel,
        mesh=mesh,
        compiler_params=pltpu.CompilerParams(
            use_tc_tiling_on_sc=False, needs_layout_passes=False
        ),
        out_type=jax.ShapeDtypeStruct((N_EDGES * DIM,), jnp.float32),
        scratch_types=[
            pltpu.VMEM((_B_PER_W,), jnp.int32),
            pltpu.VMEM((_CHUNK * DIM,), jnp.float32),
            pltpu.VMEM((_CHUNK * DIM,), jnp.float32),
            pltpu.VMEM((NUM_REL * DIM,), jnp.float32),
            pltpu.SemaphoreType.DMA,
            pltpu.SemaphoreType.DMA,
        ],
    )
    def k(idx_hbm, table_hbm, out_hbm, idx_v, rows0, rows1, table_v, sw0, sw1):
        wid = lax.axis_index("s") * _NC + lax.axis_index("c")
        base = wid * _B_PER_W
        rows = (rows0, rows1)
        sw = (sw0, sw1)

        pltpu.sync_copy(table_hbm, table_v)
        pltpu.sync_copy(idx_hbm.at[pl.ds(base, _B_PER_W)], idx_v)
        lane64 = lax.iota(jnp.int32, 16) * DIM

        def compute_chunk(i, rows_ref):
            def group(local_start):
                idxv = idx_v[pl.ds(i * _CHUNK + local_start, 16)]
                row_base = idxv << 6
                dstb = lane64 + local_start * DIM
                for c0 in range(0, DIM, 8):
                    vals = [
                        plsc.load_gather(table_v, [row_base + (c0 + u)])
                        for u in range(8)
                    ]
                    for u in range(8):
                        plsc.store_scatter(rows_ref, [dstb + (c0 + u)], vals[u])

            def body(j, carry):
                group(j * 16)
                return carry

            lax.fori_loop(0, _FULL_GROUPS, body, 0)
            group(_TAIL_START)

        def writeback(i, b):
            off = (base + i * _CHUNK) * DIM
            pltpu.async_copy(rows[b], out_hbm.at[pl.ds(off, _CHUNK * DIM)],
                             sw[b])

        def drain(i, b):
            off = (base + i * _CHUNK) * DIM
            pltpu.make_async_copy(
                rows[b], out_hbm.at[pl.ds(off, _CHUNK * DIM)], sw[b]
            ).wait()

        compute_chunk(0, rows0)
        writeback(0, 0)

        def pair(it, carry):
            i1 = 1 + 2 * it
            i0 = 2 + 2 * it

            @pl.when(it > 0)
            def _():
                drain(i1 - 2, 1)

            compute_chunk(i1, rows1)
            writeback(i1, 1)
            drain(i0 - 2, 0)
            compute_chunk(i0, rows0)
            writeback(i0, 0)
            return carry

        lax.fori_loop(0, _N_PAIRS, pair, 0)
        drain(_N_STEPS - 2, 1)
        drain(_N_STEPS - 1, 0)

    return k


_sc_kernel = _make_sc_kernel()


def kernel(relation_indices, W):
    idx = relation_indices.astype(jnp.int32)
    flat = _sc_kernel(idx, jnp.reshape(W, (-1,)))
    return jnp.reshape(flat, (N_EDGES, DIM))


# speedup vs baseline: 3.8225x; 2.8713x over previous
import functools

import jax
import jax.numpy as jnp
from jax import lax
from jax.experimental import pallas as pl
from jax.experimental.pallas import tpu as pltpu
from jax.experimental.pallas import tpu_sc as plsc

NUM_REL = 16
DIM = 64
N_EDGES = 800000

_info = plsc.get_sparse_core_info()
_NC, _NS = _info.num_cores, _info.num_subcores
_NW = _NC * _NS
_B_PER_W = N_EDGES // _NW
_CHUNK = 200
_N_STEPS = _B_PER_W // _CHUNK
_FULL_GROUPS = _CHUNK // 16
_TAIL_START = _CHUNK - 16
_N_PAIRS = (_N_STEPS - 1) // 2


def _make_sc_kernel():
    mesh = plsc.VectorSubcoreMesh(core_axis_name="c", subcore_axis_name="s")

    @functools.partial(
        pl.kernel,
        mesh=mesh,
        compiler_params=pltpu.CompilerParams(
            use_tc_tiling_on_sc=False, needs_layout_passes=False
        ),
        out_type=jax.ShapeDtypeStruct((N_EDGES * DIM,), jnp.float32),
        scratch_types=[
            pltpu.VMEM((_B_PER_W,), jnp.int32),
            pltpu.VMEM((_CHUNK * DIM,), jnp.float32),
            pltpu.VMEM((_CHUNK * DIM,), jnp.float32),
            pltpu.VMEM((NUM_REL * DIM,), jnp.float32),
            pltpu.SemaphoreType.DMA,
            pltpu.SemaphoreType.DMA,
        ],
    )
    def k(idx_hbm, table_hbm, out_hbm, idx_v, rows0, rows1, table_v, sw0, sw1):
        wid = lax.axis_index("s") * _NC + lax.axis_index("c")
        base = wid * _B_PER_W
        rows = (rows0, rows1)
        sw = (sw0, sw1)

        pltpu.sync_copy(table_hbm, table_v)
        pltpu.sync_copy(idx_hbm.at[pl.ds(base, _B_PER_W)], idx_v)

        def compute_chunk(i, rows_ref):
            def group(local_start):
                idxv = idx_v[pl.ds(i * _CHUNK + local_start, 16)]
                srcs = idxv << 6
                for u in range(0, 16, 2):
                    s0 = srcs[u]
                    s1 = srcs[u + 1]
                    vals = (
                        [table_v[pl.ds(s0 + 16 * h, 16)] for h in range(4)]
                        + [table_v[pl.ds(s1 + 16 * h, 16)] for h in range(4)]
                    )
                    d0 = (local_start + u) * DIM
                    for h in range(4):
                        rows_ref[pl.ds(d0 + 16 * h, 16)] = vals[h]
                    for h in range(4):
                        rows_ref[pl.ds(d0 + DIM + 16 * h, 16)] = vals[4 + h]

            def body(j, carry):
                group(j * 16)
                return carry

            lax.fori_loop(0, _FULL_GROUPS, body, 0)
            group(_TAIL_START)

        def writeback(i, b):
            off = (base + i * _CHUNK) * DIM
            pltpu.async_copy(rows[b], out_hbm.at[pl.ds(off, _CHUNK * DIM)],
                             sw[b])

        def drain(i, b):
            off = (base + i * _CHUNK) * DIM
            pltpu.make_async_copy(
                rows[b], out_hbm.at[pl.ds(off, _CHUNK * DIM)], sw[b]
            ).wait()

        compute_chunk(0, rows0)
        writeback(0, 0)

        def pair(it, carry):
            i1 = 1 + 2 * it
            i0 = 2 + 2 * it

            @pl.when(it > 0)
            def _():
                drain(i1 - 2, 1)

            compute_chunk(i1, rows1)
            writeback(i1, 1)
            drain(i0 - 2, 0)
            compute_chunk(i0, rows0)
            writeback(i0, 0)
            return carry

        lax.fori_loop(0, _N_PAIRS, pair, 0)
        drain(_N_STEPS - 2, 1)
        drain(_N_STEPS - 1, 0)

    return k


_sc_kernel = _make_sc_kernel()


def kernel(relation_indices, W):
    idx = relation_indices.astype(jnp.int32)
    flat = _sc_kernel(idx, jnp.reshape(W, (-1,)))
    return jnp.reshape(flat, (N_EDGES, DIM))


# speedup vs baseline: 5.6027x; 1.4657x over previous
import functools

import jax
import jax.numpy as jnp
from jax import lax
from jax.experimental import pallas as pl
from jax.experimental.pallas import tpu as pltpu
from jax.experimental.pallas import tpu_sc as plsc

NUM_REL = 16
DIM = 64
N_EDGES = 800000

_info = plsc.get_sparse_core_info()
_NC, _NS = _info.num_cores, _info.num_subcores
_NW = _NC * _NS
_B_PER_W = N_EDGES // _NW
_CHUNK = 200
_N_STEPS = _B_PER_W // _CHUNK
_FULL_GROUPS = _CHUNK // 16
_TAIL_START = _CHUNK - 16
_N_PAIRS = (_N_STEPS - 1) // 2


def _make_sc_kernel():
    mesh = plsc.VectorSubcoreMesh(core_axis_name="c", subcore_axis_name="s")

    @functools.partial(
        pl.kernel,
        mesh=mesh,
        out_type=jax.ShapeDtypeStruct((N_EDGES, DIM), jnp.float32),
        scratch_types=[
            pltpu.VMEM((_B_PER_W,), jnp.int32),
            pltpu.VMEM((_CHUNK, DIM), jnp.float32),
            pltpu.VMEM((_CHUNK, DIM), jnp.float32),
            pltpu.VMEM((NUM_REL, DIM), jnp.float32),
            pltpu.SemaphoreType.DMA,
            pltpu.SemaphoreType.DMA,
        ],
    )
    def k(idx_hbm, table_hbm, out_hbm, idx_v, rows0, rows1, table_v, sw0, sw1):
        wid = lax.axis_index("s") * _NC + lax.axis_index("c")
        base = wid * _B_PER_W
        rows = (rows0, rows1)
        sw = (sw0, sw1)

        pltpu.sync_copy(table_hbm, table_v)
        pltpu.sync_copy(idx_hbm.at[pl.ds(base, _B_PER_W)], idx_v)

        def compute_chunk(i, rows_ref):
            def group(local_start):
                idxv = idx_v[pl.ds(i * _CHUNK + local_start, 16)]
                for u in range(0, 16, 2):
                    r0 = idxv[u]
                    r1 = idxv[u + 1]
                    vals = (
                        [table_v[r0, pl.ds(16 * h, 16)] for h in range(4)]
                        + [table_v[r1, pl.ds(16 * h, 16)] for h in range(4)]
                    )
                    e0 = local_start + u
                    for h in range(4):
                        rows_ref[e0, pl.ds(16 * h, 16)] = vals[h]
                    for h in range(4):
                        rows_ref[e0 + 1, pl.ds(16 * h, 16)] = vals[4 + h]

            def body(j, carry):
                group(j * 16)
                return carry

            lax.fori_loop(0, _FULL_GROUPS, body, 0)
            group(_TAIL_START)

        def writeback(i, b):
            off = base + i * _CHUNK
            pltpu.async_copy(rows[b], out_hbm.at[pl.ds(off, _CHUNK)], sw[b])

        def drain(i, b):
            off = base + i * _CHUNK
            pltpu.make_async_copy(
                rows[b], out_hbm.at[pl.ds(off, _CHUNK)], sw[b]
            ).wait()

        compute_chunk(0, rows0)
        writeback(0, 0)

        def pair(it, carry):
            i1 = 1 + 2 * it
            i0 = 2 + 2 * it

            @pl.when(it > 0)
            def _():
                drain(i1 - 2, 1)

            compute_chunk(i1, rows1)
            writeback(i1, 1)
            drain(i0 - 2, 0)
            compute_chunk(i0, rows0)
            writeback(i0, 0)
            return carry

        lax.fori_loop(0, _N_PAIRS, pair, 0)
        drain(_N_STEPS - 2, 1)
        drain(_N_STEPS - 1, 0)

    return k


_sc_kernel = _make_sc_kernel()


def kernel(relation_indices, W):
    idx = relation_indices.astype(jnp.int32)
    return _sc_kernel(idx, W)


# speedup vs baseline: 17.0519x; 3.0435x over previous
import functools

import jax
import jax.numpy as jnp
from jax import lax
from jax.experimental import pallas as pl
from jax.experimental.pallas import tpu as pltpu
from jax.experimental.pallas import tpu_sc as plsc

NUM_REL = 16
DIM = 64
N_EDGES = 800000

_info = plsc.get_sparse_core_info()
_NC, _NS = _info.num_cores, _info.num_subcores
_NW = _NC * _NS
_N_TILES = N_EDGES // 128
_T_BASE = _N_TILES // _NW
_T_EXTRA = _N_TILES % _NW
_TPC = 4
_ECH = 128 * _TPC
_N_CHUNKS = -(-(_T_BASE + 1) // _TPC)
_N_PAIRS = (_N_CHUNKS - 1) // 2


def _make_sc_kernel():
    mesh = plsc.VectorSubcoreMesh(core_axis_name="c", subcore_axis_name="s")

    @functools.partial(
        pl.kernel,
        mesh=mesh,
        compiler_params=pltpu.CompilerParams(needs_layout_passes=False),
        out_type=jax.ShapeDtypeStruct((DIM, N_EDGES), jnp.float32),
        scratch_types=[
            pltpu.VMEM((2, _ECH), jnp.int32),
            pltpu.VMEM((DIM, _ECH), jnp.float32),
            pltpu.VMEM((DIM, _ECH), jnp.float32),
            pltpu.VMEM((NUM_REL * DIM,), jnp.float32),
            pltpu.SemaphoreType.DMA,
            pltpu.SemaphoreType.DMA,
            pltpu.SemaphoreType.DMA,
            pltpu.SemaphoreType.DMA,
        ],
    )
    def k(idx_hbm, table_hbm, out_hbm, idx_v, buf0, buf1, table_v,
          si0, si1, sw0, sw1):
        wid = lax.axis_index("s") * _NC + lax.axis_index("c")
        ts = wid * _T_BASE + jnp.minimum(wid, _T_EXTRA)
        nt = _T_BASE + jnp.where(wid < _T_EXTRA, 1, 0)
        bufs = (buf0, buf1)
        si = (si0, si1)
        sw = (sw0, sw1)

        pltpu.sync_copy(table_hbm, table_v)

        def chunk_e0(kk):
            return 128 * (ts + jnp.minimum(kk * _TPC, nt - _TPC))

        def stage_idx(kk, b):
            pltpu.async_copy(
                idx_hbm.at[pl.ds(chunk_e0(kk), _ECH)], idx_v.at[b], si[b]
            )

        def wait_idx(kk, b):
            pltpu.make_async_copy(
                idx_hbm.at[pl.ds(chunk_e0(kk), _ECH)], idx_v.at[b], si[b]
            ).wait()

        def compute_chunk(b):
            buf = bufs[b]

            def group(eg, carry):
                e0 = eg * 16
                idxv = idx_v[b, pl.ds(e0, 16)]
                for c0 in range(0, DIM, 8):
                    vals = [
                        plsc.load_gather(table_v, [idxv + (c0 + u) * 16])
                        for u in range(8)
                    ]
                    for u in range(8):
                        buf[c0 + u, pl.ds(e0, 16)] = vals[u]
                return carry

            lax.fori_loop(0, _ECH // 16, group, 0)

        def writeback(kk, b):
            e0 = chunk_e0(kk)
            pltpu.async_copy(
                bufs[b], out_hbm.at[:, pl.ds(e0, _ECH)], sw[b]
            )

        def drain(kk, b):
            e0 = chunk_e0(kk)
            pltpu.make_async_copy(
                bufs[b], out_hbm.at[:, pl.ds(e0, _ECH)], sw[b]
            ).wait()

        stage_idx(0, 0)
        wait_idx(0, 0)
        stage_idx(1, 1)
        compute_chunk(0)
        writeback(0, 0)

        def pair(it, carry):
            k1 = 1 + 2 * it
            k0 = 2 + 2 * it
            wait_idx(k1, 1)
            stage_idx(k0, 0)

            @pl.when(it > 0)
            def _():
                drain(k1 - 2, 1)

            compute_chunk(1)
            writeback(k1, 1)

            wait_idx(k0, 0)

            @pl.when(k0 + 1 < _N_CHUNKS)
            def _():
                stage_idx(k0 + 1, 1)

            drain(k0 - 2, 0)
            compute_chunk(0)
            writeback(k0, 0)
            return carry

        lax.fori_loop(0, _N_PAIRS, pair, 0)
        drain(_N_CHUNKS - 2, 1)
        drain(_N_CHUNKS - 1, 0)

    return k


_sc_kernel = _make_sc_kernel()


def kernel(relation_indices, W):
    idx = relation_indices.astype(jnp.int32)
    table_t = jnp.reshape(jnp.transpose(W), (-1,))
    out_t = _sc_kernel(idx, table_t)
    return jnp.transpose(out_t)
